# Initial kernel scaffold; baseline (speedup 1.0000x reference)
import jax
import jax.numpy as jnp
from jax.experimental import pallas as pl


def kernel(x, edge_index, edge_attr, W_msg, b_msg, W_upd, b_upd):
    raise NotImplementedError("write your pallas kernel here")


# speedup vs baseline: 3.6803x; 3.6803x over previous
import functools

import jax
import jax.numpy as jnp
from jax import lax
from jax.experimental import pallas as pl
from jax.experimental.pallas import tpu as pltpu
from jax.experimental.pallas import tpu_sc as plsc

N = 10000
E = 320000
D = 128
DE = 16

NC = 2
NS = 16
NW = NC * NS
EPW = E // NW
K = 80
NBLK = EPW // K
STRIPE = 624
TAIL = N - NS * STRIPE


def _prep_body(x_ref, w1_ref, w2_ref, b_ref, a_ref, bout_ref):
    xb = x_ref[...]
    a_ref[...] = jnp.dot(xb, w1_ref[...], preferred_element_type=jnp.float32)
    bout_ref[...] = (
        jnp.dot(xb, w2_ref[...], preferred_element_type=jnp.float32)
        + b_ref[...]
    )


def _node_prep(x, w1, w2, b):
    bn = 2000
    return pl.pallas_call(
        _prep_body,
        grid=(N // bn,),
        in_specs=[
            pl.BlockSpec((bn, D), lambda i: (i, 0)),
            pl.BlockSpec((D, D), lambda i: (0, 0)),
            pl.BlockSpec((D, D), lambda i: (0, 0)),
            pl.BlockSpec((1, D), lambda i: (0, 0)),
        ],
        out_specs=[
            pl.BlockSpec((bn, D), lambda i: (i, 0)),
            pl.BlockSpec((bn, D), lambda i: (i, 0)),
        ],
        out_shape=[
            jax.ShapeDtypeStruct((N, D), jnp.float32),
            jax.ShapeDtypeStruct((N, D), jnp.float32),
        ],
    )(x, w1, w2, b.reshape(1, D))


def _cmsg_body(ea_ref, w3_ref, c_ref):
    c_ref[...] = jnp.dot(ea_ref[...], w3_ref[...],
                         preferred_element_type=jnp.float32)


def _edge_prep(ea, w3):
    be = 4000
    return pl.pallas_call(
        _cmsg_body,
        grid=(E // be,),
        in_specs=[
            pl.BlockSpec((be, DE), lambda i: (i, 0)),
            pl.BlockSpec((DE, D), lambda i: (0, 0)),
        ],
        out_specs=pl.BlockSpec((be, D), lambda i: (i, 0)),
        out_shape=jax.ShapeDtypeStruct((E, D), jnp.float32),
    )(ea, w3)


def _update_body(x_ref, g0_ref, g1_ref, wu1_ref, wu2_ref, b_ref, o_ref):
    xb = x_ref[...]
    agg = g0_ref[...] + g1_ref[...]
    h = (
        jnp.dot(xb, wu1_ref[...], preferred_element_type=jnp.float32)
        + jnp.dot(agg, wu2_ref[...], preferred_element_type=jnp.float32)
        + b_ref[...]
    )
    o_ref[...] = xb + jnp.maximum(h, 0.0)


def _node_update(x, g0, g1, wu1, wu2, b):
    bn = 2000
    return pl.pallas_call(
        _update_body,
        grid=(N // bn,),
        in_specs=[
            pl.BlockSpec((bn, D), lambda i: (i, 0)),
            pl.BlockSpec((bn, D), lambda i: (i, 0)),
            pl.BlockSpec((bn, D), lambda i: (i, 0)),
            pl.BlockSpec((D, D), lambda i: (0, 0)),
            pl.BlockSpec((D, D), lambda i: (0, 0)),
            pl.BlockSpec((1, D), lambda i: (0, 0)),
        ],
        out_specs=pl.BlockSpec((bn, D), lambda i: (i, 0)),
        out_shape=jax.ShapeDtypeStruct((N, D), jnp.float32),
    )(x, g0, g1, wu1, wu2, b.reshape(1, D))


def _sc_edges(a, b, c, senders, receivers):
    mesh = plsc.VectorSubcoreMesh(core_axis_name="c", subcore_axis_name="s")

    @functools.partial(
        pl.kernel,
        mesh=mesh,
        out_type=jax.ShapeDtypeStruct((NC, N, D), jnp.float32),
        scratch_types=[
            pltpu.VMEM((K,), jnp.int32),
            pltpu.VMEM((K,), jnp.int32),
            pltpu.VMEM((K, D), jnp.float32),
            pltpu.VMEM((K, D), jnp.float32),
            pltpu.VMEM((K, D), jnp.float32),
            pltpu.VMEM_SHARED((N, D), jnp.float32),
            pltpu.SemaphoreType.DMA,
            pltpu.SemaphoreType.DMA,
            pltpu.SemaphoreType.DMA,
        ],
    )
    def sc_kernel(a_hbm, b_hbm, c_hbm, s_hbm, r_hbm, out_hbm,
                  sidx, ridx, av, bv, cv, agg, sem_a, sem_b, sem_c):
        cid = lax.axis_index("c")
        sid = lax.axis_index("s")
        wid = sid * NC + cid

        @pl.loop(0, K)
        def _(i):
            for j in range(D // 16):
                av[i, pl.ds(j * 16, 16)] = jnp.zeros((16,), jnp.float32)

        base_row = sid * STRIPE
        full, rem = divmod(STRIPE, K)

        @pl.loop(0, full)
        def _(t):
            pltpu.sync_copy(av, agg.at[pl.ds(base_row + t * K, K)])

        if rem:
            pltpu.sync_copy(av.at[pl.ds(0, rem)],
                            agg.at[pl.ds(base_row + full * K, rem)])

        @pl.when(sid == NS - 1)
        def _():
            pltpu.sync_copy(av.at[pl.ds(0, TAIL)],
                            agg.at[pl.ds(NS * STRIPE, TAIL)])

        plsc.subcore_barrier()

        @pl.loop(0, NBLK)
        def _(t):
            ebase = wid * EPW + t * K
            pltpu.sync_copy(s_hbm.at[pl.ds(ebase, K)], sidx)
            pltpu.sync_copy(r_hbm.at[pl.ds(ebase, K)], ridx)
            ga = pltpu.async_copy(a_hbm.at[sidx], av, sem_a)
            gb = pltpu.async_copy(b_hbm.at[ridx], bv, sem_b)
            gc = pltpu.async_copy(c_hbm.at[pl.ds(ebase, K)], cv, sem_c)
            ga.wait()
            gb.wait()
            gc.wait()

            @pl.loop(0, K)
            def _(i):
                for j in range(D // 16):
                    sl = pl.ds(j * 16, 16)
                    av[i, sl] = jnp.maximum(av[i, sl] + bv[i, sl] + cv[i, sl],
                                            0.0)

            pltpu.sync_copy(av, agg.at[ridx], add=True)

        plsc.subcore_barrier()

        pltpu.sync_copy(agg.at[pl.ds(base_row, STRIPE)],
                        out_hbm.at[cid, pl.ds(base_row, STRIPE)])

        @pl.when(sid == NS - 1)
        def _():
            pltpu.sync_copy(agg.at[pl.ds(NS * STRIPE, TAIL)],
                            out_hbm.at[cid, pl.ds(NS * STRIPE, TAIL)])

    return sc_kernel(a, b, c, senders, receivers)


@jax.jit
def kernel(x, edge_index, edge_attr, W_msg, b_msg, W_upd, b_upd):
    assert x.shape == (N, D) and edge_attr.shape == (E, DE)
    w1 = W_msg[:D]
    w2 = W_msg[D:2 * D]
    w3 = W_msg[2 * D:]
    senders = edge_index[0]
    receivers = edge_index[1]

    a, b = _node_prep(x, w1, w2, b_msg)
    c = _edge_prep(edge_attr, w3)
    partials = _sc_edges(a, b, c, senders, receivers)
    return _node_update(x, partials[0], partials[1],
                        W_upd[:D], W_upd[D:], b_upd)


# speedup vs baseline: 4.2735x; 1.1612x over previous
import functools

import jax
import jax.numpy as jnp
from jax import lax
from jax.experimental import pallas as pl
from jax.experimental.pallas import tpu as pltpu
from jax.experimental.pallas import tpu_sc as plsc

N = 10000
E = 320000
D = 128
DE = 16

NC = 2
NS = 16
NW = NC * NS
EPW = E // NW
K = 40
NBLK = EPW // K
STRIPE = 624
TAIL = N - NS * STRIPE


def _prep_body(x_ref, w1_ref, w2_ref, b_ref, a_ref, bout_ref):
    xb = x_ref[...]
    a_ref[...] = jnp.dot(xb, w1_ref[...], preferred_element_type=jnp.float32)
    bout_ref[...] = (
        jnp.dot(xb, w2_ref[...], preferred_element_type=jnp.float32)
        + b_ref[...]
    )


def _node_prep(x, w1, w2, b):
    bn = 2000
    return pl.pallas_call(
        _prep_body,
        grid=(N // bn,),
        in_specs=[
            pl.BlockSpec((bn, D), lambda i: (i, 0)),
            pl.BlockSpec((D, D), lambda i: (0, 0)),
            pl.BlockSpec((D, D), lambda i: (0, 0)),
            pl.BlockSpec((1, D), lambda i: (0, 0)),
        ],
        out_specs=[
            pl.BlockSpec((bn, D), lambda i: (i, 0)),
            pl.BlockSpec((bn, D), lambda i: (i, 0)),
        ],
        out_shape=[
            jax.ShapeDtypeStruct((N, D), jnp.float32),
            jax.ShapeDtypeStruct((N, D), jnp.float32),
        ],
    )(x, w1, w2, b.reshape(1, D))


def _cmsg_body(ea_ref, w3_ref, c_ref):
    c_ref[...] = jnp.dot(ea_ref[...], w3_ref[...],
                         preferred_element_type=jnp.float32)


def _edge_prep(ea, w3):
    be = 4000
    return pl.pallas_call(
        _cmsg_body,
        grid=(E // be,),
        in_specs=[
            pl.BlockSpec((be, DE), lambda i: (i, 0)),
            pl.BlockSpec((DE, D), lambda i: (0, 0)),
        ],
        out_specs=pl.BlockSpec((be, D), lambda i: (i, 0)),
        out_shape=jax.ShapeDtypeStruct((E, D), jnp.float32),
    )(ea, w3)


def _update_body(x_ref, g0_ref, g1_ref, wu1_ref, wu2_ref, b_ref, o_ref):
    xb = x_ref[...]
    agg = g0_ref[...] + g1_ref[...]
    h = (
        jnp.dot(xb, wu1_ref[...], preferred_element_type=jnp.float32)
        + jnp.dot(agg, wu2_ref[...], preferred_element_type=jnp.float32)
        + b_ref[...]
    )
    o_ref[...] = xb + jnp.maximum(h, 0.0)


def _node_update(x, g0, g1, wu1, wu2, b):
    bn = 2000
    return pl.pallas_call(
        _update_body,
        grid=(N // bn,),
        in_specs=[
            pl.BlockSpec((bn, D), lambda i: (i, 0)),
            pl.BlockSpec((bn, D), lambda i: (i, 0)),
            pl.BlockSpec((bn, D), lambda i: (i, 0)),
            pl.BlockSpec((D, D), lambda i: (0, 0)),
            pl.BlockSpec((D, D), lambda i: (0, 0)),
            pl.BlockSpec((1, D), lambda i: (0, 0)),
        ],
        out_specs=pl.BlockSpec((bn, D), lambda i: (i, 0)),
        out_shape=jax.ShapeDtypeStruct((N, D), jnp.float32),
    )(x, g0, g1, wu1, wu2, b.reshape(1, D))


def _sc_edges(a, b, c, senders, receivers):
    mesh = plsc.VectorSubcoreMesh(core_axis_name="c", subcore_axis_name="s")
    @functools.partial(
        pl.kernel,
        mesh=mesh,
        out_type=jax.ShapeDtypeStruct((NC, N, D), jnp.float32),
        scratch_types=[
            pltpu.VMEM((2, K), jnp.int32),
            pltpu.VMEM((2, K), jnp.int32),
            pltpu.VMEM((2, K, D), jnp.float32),
            pltpu.VMEM((2, K, D), jnp.float32),
            pltpu.VMEM((2, K, D), jnp.float32),
            pltpu.VMEM_SHARED((N, D), jnp.float32),
            pltpu.SemaphoreType.DMA,
            pltpu.SemaphoreType.DMA,
            pltpu.SemaphoreType.DMA,
        ],
    )
    def sc_kernel(a_hbm, b_hbm, c_hbm, s_hbm, r_hbm, out_hbm,
                  sidx, ridx, av2, bv2, cv2, agg, sem_a, sem_b, sem_c):
        cid = lax.axis_index("c")
        sid = lax.axis_index("s")
        wid = sid * NC + cid

        z = av2.at[0]

        @pl.loop(0, K)
        def _(i):
            for j in range(D // 16):
                z[i, pl.ds(j * 16, 16)] = jnp.zeros((16,), jnp.float32)

        base_row = sid * STRIPE
        full, rem = divmod(STRIPE, K)

        @pl.loop(0, full)
        def _(t):
            pltpu.sync_copy(z, agg.at[pl.ds(base_row + t * K, K)])

        if rem:
            pltpu.sync_copy(z.at[pl.ds(0, rem)],
                            agg.at[pl.ds(base_row + full * K, rem)])

        @pl.when(sid == NS - 1)
        def _():
            pltpu.sync_copy(z.at[pl.ds(0, TAIL)],
                            agg.at[pl.ds(NS * STRIPE, TAIL)])

        plsc.subcore_barrier()

        def issue(t, bf):
            ebase = wid * EPW + t * K
            pltpu.sync_copy(s_hbm.at[pl.ds(ebase, K)], sidx.at[bf])
            pltpu.sync_copy(r_hbm.at[pl.ds(ebase, K)], ridx.at[bf])
            pltpu.async_copy(a_hbm.at[sidx.at[bf]], av2.at[bf], sem_a)
            pltpu.async_copy(b_hbm.at[ridx.at[bf]], bv2.at[bf], sem_b)
            pltpu.async_copy(c_hbm.at[pl.ds(ebase, K)], cv2.at[bf], sem_c)

        def drain(t, bf):
            ebase = wid * EPW + t * K
            pltpu.make_async_copy(a_hbm.at[sidx.at[bf]], av2.at[bf],
                                  sem_a).wait()
            pltpu.make_async_copy(b_hbm.at[ridx.at[bf]], bv2.at[bf],
                                  sem_b).wait()
            pltpu.make_async_copy(c_hbm.at[pl.ds(ebase, K)], cv2.at[bf],
                                  sem_c).wait()

        issue(0, 0)

        @pl.loop(0, NBLK, step=2)
        def _(t0):
            for bf in range(2):
                t = t0 + bf

                @pl.when(t < NBLK)
                def _():
                    @pl.when(t + 1 < NBLK)
                    def _():
                        issue(t + 1, 1 - bf)

                    drain(t, bf)
                    m = av2.at[bf]
                    bb = bv2.at[bf]
                    cc = cv2.at[bf]

                    @pl.loop(0, K)
                    def _(i):
                        for j in range(D // 16):
                            sl = pl.ds(j * 16, 16)
                            m[i, sl] = jnp.maximum(
                                m[i, sl] + bb[i, sl] + cc[i, sl], 0.0)

                    pltpu.sync_copy(m, agg.at[ridx.at[bf]], add=True)

        plsc.subcore_barrier()

        pltpu.sync_copy(agg.at[pl.ds(base_row, STRIPE)],
                        out_hbm.at[cid, pl.ds(base_row, STRIPE)])

        @pl.when(sid == NS - 1)
        def _():
            pltpu.sync_copy(agg.at[pl.ds(NS * STRIPE, TAIL)],
                            out_hbm.at[cid, pl.ds(NS * STRIPE, TAIL)])

    return sc_kernel(a, b, c, senders, receivers)


@jax.jit
def kernel(x, edge_index, edge_attr, W_msg, b_msg, W_upd, b_upd):
    assert x.shape == (N, D) and edge_attr.shape == (E, DE)
    w1 = W_msg[:D]
    w2 = W_msg[D:2 * D]
    w3 = W_msg[2 * D:]
    senders = edge_index[0]
    receivers = edge_index[1]

    a, b = _node_prep(x, w1, w2, b_msg)
    c = _edge_prep(edge_attr, w3)
    partials = _sc_edges(a, b, c, senders, receivers)
    return _node_update(x, partials[0], partials[1],
                        W_upd[:D], W_upd[D:], b_upd)


# speedup vs baseline: 4.4468x; 1.0405x over previous
import functools

import jax
import jax.numpy as jnp
from jax import lax
from jax.experimental import pallas as pl
from jax.experimental.pallas import tpu as pltpu
from jax.experimental.pallas import tpu_sc as plsc

N = 10000
E = 320000
D = 128
DE = 16

NC = 2
NS = 16
NW = NC * NS
EPW = E // NW
K = 40
NBLK = EPW // K
STRIPE = 624
TAIL = N - NS * STRIPE


def _prep_body(x_ref, w1_ref, w2_ref, b_ref, a_ref, bout_ref):
    xb = x_ref[...]
    a_ref[...] = jnp.dot(xb, w1_ref[...], preferred_element_type=jnp.float32)
    bout_ref[...] = (
        jnp.dot(xb, w2_ref[...], preferred_element_type=jnp.float32)
        + b_ref[...]
    )


def _node_prep(x, w1, w2, b):
    bn = 2000
    return pl.pallas_call(
        _prep_body,
        grid=(N // bn,),
        in_specs=[
            pl.BlockSpec((bn, D), lambda i: (i, 0)),
            pl.BlockSpec((D, D), lambda i: (0, 0)),
            pl.BlockSpec((D, D), lambda i: (0, 0)),
            pl.BlockSpec((1, D), lambda i: (0, 0)),
        ],
        out_specs=[
            pl.BlockSpec((bn, D), lambda i: (i, 0)),
            pl.BlockSpec((bn, D), lambda i: (i, 0)),
        ],
        out_shape=[
            jax.ShapeDtypeStruct((N, D), jnp.float32),
            jax.ShapeDtypeStruct((N, D), jnp.float32),
        ],
    )(x, w1, w2, b.reshape(1, D))


def _cmsg_body(ea_ref, w3_ref, c_ref):
    c_ref[...] = jnp.dot(ea_ref[...], w3_ref[...],
                         preferred_element_type=jnp.float32)


def _edge_prep(ea, w3):
    P = 8
    be = 2000
    ea8 = ea.reshape(E // P, P * DE)
    w3p = jnp.kron(jnp.eye(P, dtype=jnp.float32), w3)
    c8 = pl.pallas_call(
        _cmsg_body,
        grid=(E // P // be,),
        in_specs=[
            pl.BlockSpec((be, P * DE), lambda i: (i, 0)),
            pl.BlockSpec((P * DE, P * D), lambda i: (0, 0)),
        ],
        out_specs=pl.BlockSpec((be, P * D), lambda i: (i, 0)),
        out_shape=jax.ShapeDtypeStruct((E // P, P * D), jnp.float32),
    )(ea8, w3p)
    return c8.reshape(E, D)


def _update_body(x_ref, g0_ref, g1_ref, wu1_ref, wu2_ref, b_ref, o_ref):
    xb = x_ref[...]
    agg = g0_ref[...] + g1_ref[...]
    h = (
        jnp.dot(xb, wu1_ref[...], preferred_element_type=jnp.float32)
        + jnp.dot(agg, wu2_ref[...], preferred_element_type=jnp.float32)
        + b_ref[...]
    )
    o_ref[...] = xb + jnp.maximum(h, 0.0)


def _node_update(x, g0, g1, wu1, wu2, b):
    bn = 2000
    return pl.pallas_call(
        _update_body,
        grid=(N // bn,),
        in_specs=[
            pl.BlockSpec((bn, D), lambda i: (i, 0)),
            pl.BlockSpec((bn, D), lambda i: (i, 0)),
            pl.BlockSpec((bn, D), lambda i: (i, 0)),
            pl.BlockSpec((D, D), lambda i: (0, 0)),
            pl.BlockSpec((D, D), lambda i: (0, 0)),
            pl.BlockSpec((1, D), lambda i: (0, 0)),
        ],
        out_specs=pl.BlockSpec((bn, D), lambda i: (i, 0)),
        out_shape=jax.ShapeDtypeStruct((N, D), jnp.float32),
    )(x, g0, g1, wu1, wu2, b.reshape(1, D))


def _sc_edges(a, b, c, senders, receivers):
    mesh = plsc.VectorSubcoreMesh(core_axis_name="c", subcore_axis_name="s")
    @functools.partial(
        pl.kernel,
        mesh=mesh,
        out_type=jax.ShapeDtypeStruct((NC, N, D), jnp.float32),
        scratch_types=[
            pltpu.VMEM((4, K), jnp.int32),
            pltpu.VMEM((4, K), jnp.int32),
            pltpu.VMEM((2, K, D), jnp.float32),
            pltpu.VMEM((2, K, D), jnp.float32),
            pltpu.VMEM((2, K, D), jnp.float32),
            pltpu.VMEM_SHARED((N, D), jnp.float32),
            pltpu.SemaphoreType.DMA,
            pltpu.SemaphoreType.DMA,
            pltpu.SemaphoreType.DMA,
            pltpu.SemaphoreType.DMA,
            pltpu.SemaphoreType.DMA,
        ],
    )
    def sc_kernel(a_hbm, b_hbm, c_hbm, s_hbm, r_hbm, out_hbm,
                  sidx, ridx, av2, bv2, cv2, agg,
                  sem_a, sem_b, sem_c, sem_si, sem_ri):
        cid = lax.axis_index("c")
        sid = lax.axis_index("s")
        wid = sid * NC + cid

        z = av2.at[0]

        @pl.loop(0, K)
        def _(i):
            for j in range(D // 16):
                z[i, pl.ds(j * 16, 16)] = jnp.zeros((16,), jnp.float32)

        base_row = sid * STRIPE
        full, rem = divmod(STRIPE, K)

        @pl.loop(0, full)
        def _(t):
            pltpu.sync_copy(z, agg.at[pl.ds(base_row + t * K, K)])

        if rem:
            pltpu.sync_copy(z.at[pl.ds(0, rem)],
                            agg.at[pl.ds(base_row + full * K, rem)])

        @pl.when(sid == NS - 1)
        def _():
            pltpu.sync_copy(z.at[pl.ds(0, TAIL)],
                            agg.at[pl.ds(NS * STRIPE, TAIL)])

        plsc.subcore_barrier()

        def issue_idx(t, slot):
            ebase = wid * EPW + t * K
            pltpu.async_copy(s_hbm.at[pl.ds(ebase, K)], sidx.at[slot],
                             sem_si)
            pltpu.async_copy(r_hbm.at[pl.ds(ebase, K)], ridx.at[slot],
                             sem_ri)

        def wait_idx(t, slot):
            ebase = wid * EPW + t * K
            pltpu.make_async_copy(s_hbm.at[pl.ds(ebase, K)], sidx.at[slot],
                                  sem_si).wait()
            pltpu.make_async_copy(r_hbm.at[pl.ds(ebase, K)], ridx.at[slot],
                                  sem_ri).wait()

        def issue(t, slot, bf):
            ebase = wid * EPW + t * K
            pltpu.async_copy(a_hbm.at[sidx.at[slot]], av2.at[bf], sem_a)
            pltpu.async_copy(b_hbm.at[ridx.at[slot]], bv2.at[bf], sem_b)
            pltpu.async_copy(c_hbm.at[pl.ds(ebase, K)], cv2.at[bf], sem_c)

        def drain(t, slot, bf):
            ebase = wid * EPW + t * K
            pltpu.make_async_copy(a_hbm.at[sidx.at[slot]], av2.at[bf],
                                  sem_a).wait()
            pltpu.make_async_copy(b_hbm.at[ridx.at[slot]], bv2.at[bf],
                                  sem_b).wait()
            pltpu.make_async_copy(c_hbm.at[pl.ds(ebase, K)], cv2.at[bf],
                                  sem_c).wait()

        issue_idx(0, 0)
        issue_idx(1, 1)
        wait_idx(0, 0)
        issue(0, 0, 0)

        @pl.loop(0, NBLK + 2, step=4)
        def _(t0):
            for u in range(4):
                t = t0 + u
                slot = u & 3
                bf = u & 1

                @pl.when(t < NBLK)
                def _():
                    @pl.when(t + 1 < NBLK)
                    def _():
                        wait_idx(t + 1, (u + 1) & 3)
                        issue(t + 1, (u + 1) & 3, (u + 1) & 1)

                    @pl.when(t + 2 < NBLK)
                    def _():
                        issue_idx(t + 2, (u + 2) & 3)

                    drain(t, slot, bf)
                    m = av2.at[bf]
                    bb = bv2.at[bf]
                    cc = cv2.at[bf]

                    @pl.loop(0, K)
                    def _(i):
                        for j in range(D // 16):
                            sl = pl.ds(j * 16, 16)
                            m[i, sl] = jnp.maximum(
                                m[i, sl] + bb[i, sl] + cc[i, sl], 0.0)

                    pltpu.sync_copy(m, agg.at[ridx.at[slot]], add=True)

        plsc.subcore_barrier()

        pltpu.sync_copy(agg.at[pl.ds(base_row, STRIPE)],
                        out_hbm.at[cid, pl.ds(base_row, STRIPE)])

        @pl.when(sid == NS - 1)
        def _():
            pltpu.sync_copy(agg.at[pl.ds(NS * STRIPE, TAIL)],
                            out_hbm.at[cid, pl.ds(NS * STRIPE, TAIL)])

    return sc_kernel(a, b, c, senders, receivers)


@jax.jit
def kernel(x, edge_index, edge_attr, W_msg, b_msg, W_upd, b_upd):
    assert x.shape == (N, D) and edge_attr.shape == (E, DE)
    w1 = W_msg[:D]
    w2 = W_msg[D:2 * D]
    w3 = W_msg[2 * D:]
    senders = edge_index[0]
    receivers = edge_index[1]

    a, b = _node_prep(x, w1, w2, b_msg)
    c = _edge_prep(edge_attr, w3)
    partials = _sc_edges(a, b, c, senders, receivers)
    return _node_update(x, partials[0], partials[1],
                        W_upd[:D], W_upd[D:], b_upd)


# speedup vs baseline: 5.4772x; 1.2317x over previous
import functools

import jax
import jax.numpy as jnp
from jax import lax
from jax.experimental import pallas as pl
from jax.experimental.pallas import tpu as pltpu
from jax.experimental.pallas import tpu_sc as plsc

N = 10000
E = 320000
D = 128
DE = 16

NC = 2
NS = 16
NW = NC * NS
EPW = E // NW
K = 40
NBLK = EPW // K
STRIPE = 624
TAIL = N - NS * STRIPE


def _prep_body(x_ref, w1_ref, w2_ref, b_ref, a_ref, bout_ref):
    xb = x_ref[...]
    a_ref[...] = jnp.dot(xb, w1_ref[...], preferred_element_type=jnp.float32)
    bout_ref[...] = (
        jnp.dot(xb, w2_ref[...], preferred_element_type=jnp.float32)
        + b_ref[...]
    )


def _node_prep(x, w1, w2, b):
    bn = 2000
    return pl.pallas_call(
        _prep_body,
        grid=(N // bn,),
        in_specs=[
            pl.BlockSpec((bn, D), lambda i: (i, 0)),
            pl.BlockSpec((D, D), lambda i: (0, 0)),
            pl.BlockSpec((D, D), lambda i: (0, 0)),
            pl.BlockSpec((1, D), lambda i: (0, 0)),
        ],
        out_specs=[
            pl.BlockSpec((bn, D), lambda i: (i, 0)),
            pl.BlockSpec((bn, D), lambda i: (i, 0)),
        ],
        out_shape=[
            jax.ShapeDtypeStruct((N, D), jnp.float32),
            jax.ShapeDtypeStruct((N, D), jnp.float32),
        ],
    )(x, w1, w2, b.reshape(1, D))


def _cmsg_body(ea_ref, w3_ref, c_ref):
    c_ref[...] = jnp.dot(ea_ref[...], w3_ref[...],
                         preferred_element_type=jnp.float32)


def _edge_prep(ea, w3):
    be = 4000
    return pl.pallas_call(
        _cmsg_body,
        grid=(E // be,),
        in_specs=[
            pl.BlockSpec((be, DE), lambda i: (i, 0)),
            pl.BlockSpec((DE, D), lambda i: (0, 0)),
        ],
        out_specs=pl.BlockSpec((be, D), lambda i: (i, 0)),
        out_shape=jax.ShapeDtypeStruct((E, D), jnp.float32),
    )(ea, w3)


def _update_body(x_ref, g0_ref, g1_ref, wu1_ref, wu2_ref, b_ref, o_ref):
    xb = x_ref[...]
    agg = g0_ref[...] + g1_ref[...]
    h = (
        jnp.dot(xb, wu1_ref[...], preferred_element_type=jnp.float32)
        + jnp.dot(agg, wu2_ref[...], preferred_element_type=jnp.float32)
        + b_ref[...]
    )
    o_ref[...] = xb + jnp.maximum(h, 0.0)


def _node_update(x, g0, g1, wu1, wu2, b):
    bn = 2000
    return pl.pallas_call(
        _update_body,
        grid=(N // bn,),
        in_specs=[
            pl.BlockSpec((bn, D), lambda i: (i, 0)),
            pl.BlockSpec((bn, D), lambda i: (i, 0)),
            pl.BlockSpec((bn, D), lambda i: (i, 0)),
            pl.BlockSpec((D, D), lambda i: (0, 0)),
            pl.BlockSpec((D, D), lambda i: (0, 0)),
            pl.BlockSpec((1, D), lambda i: (0, 0)),
        ],
        out_specs=pl.BlockSpec((bn, D), lambda i: (i, 0)),
        out_shape=jax.ShapeDtypeStruct((N, D), jnp.float32),
    )(x, g0, g1, wu1, wu2, b.reshape(1, D))


def _sc_edges(a, b, c, senders, receivers):
    mesh = plsc.VectorSubcoreMesh(core_axis_name="c", subcore_axis_name="s")
    @functools.partial(
        pl.kernel,
        mesh=mesh,
        out_type=jax.ShapeDtypeStruct((NC, N, D), jnp.float32),
        scratch_types=[
            pltpu.VMEM((4, K), jnp.int32),
            pltpu.VMEM((4, K), jnp.int32),
            pltpu.VMEM((2, K, D), jnp.float32),
            pltpu.VMEM((2, K, D), jnp.float32),
            pltpu.VMEM((2, K, D), jnp.float32),
            pltpu.VMEM_SHARED((N, D), jnp.float32),
            pltpu.SemaphoreType.DMA,
            pltpu.SemaphoreType.DMA,
            pltpu.SemaphoreType.DMA,
            pltpu.SemaphoreType.DMA,
            pltpu.SemaphoreType.DMA,
        ],
    )
    def sc_kernel(a_hbm, b_hbm, c_hbm, s_hbm, r_hbm, out_hbm,
                  sidx, ridx, av2, bv2, cv2, agg,
                  sem_a, sem_b, sem_c, sem_si, sem_ri):
        cid = lax.axis_index("c")
        sid = lax.axis_index("s")
        wid = sid * NC + cid

        z = av2.at[0]

        @pl.loop(0, K)
        def _(i):
            for j in range(D // 16):
                z[i, pl.ds(j * 16, 16)] = jnp.zeros((16,), jnp.float32)

        base_row = sid * STRIPE
        full, rem = divmod(STRIPE, K)

        @pl.loop(0, full)
        def _(t):
            pltpu.sync_copy(z, agg.at[pl.ds(base_row + t * K, K)])

        if rem:
            pltpu.sync_copy(z.at[pl.ds(0, rem)],
                            agg.at[pl.ds(base_row + full * K, rem)])

        @pl.when(sid == NS - 1)
        def _():
            pltpu.sync_copy(z.at[pl.ds(0, TAIL)],
                            agg.at[pl.ds(NS * STRIPE, TAIL)])

        plsc.subcore_barrier()

        def issue_idx(t, slot):
            ebase = wid * EPW + t * K
            pltpu.async_copy(s_hbm.at[pl.ds(ebase, K)], sidx.at[slot],
                             sem_si)
            pltpu.async_copy(r_hbm.at[pl.ds(ebase, K)], ridx.at[slot],
                             sem_ri)

        def wait_idx(t, slot):
            ebase = wid * EPW + t * K
            pltpu.make_async_copy(s_hbm.at[pl.ds(ebase, K)], sidx.at[slot],
                                  sem_si).wait()
            pltpu.make_async_copy(r_hbm.at[pl.ds(ebase, K)], ridx.at[slot],
                                  sem_ri).wait()

        def issue(t, slot, bf):
            ebase = wid * EPW + t * K
            pltpu.async_copy(a_hbm.at[sidx.at[slot]], av2.at[bf], sem_a)
            pltpu.async_copy(b_hbm.at[ridx.at[slot]], bv2.at[bf], sem_b)
            pltpu.async_copy(c_hbm.at[pl.ds(ebase, K)], cv2.at[bf], sem_c)

        def drain(t, slot, bf):
            ebase = wid * EPW + t * K
            pltpu.make_async_copy(a_hbm.at[sidx.at[slot]], av2.at[bf],
                                  sem_a).wait()
            pltpu.make_async_copy(b_hbm.at[ridx.at[slot]], bv2.at[bf],
                                  sem_b).wait()
            pltpu.make_async_copy(c_hbm.at[pl.ds(ebase, K)], cv2.at[bf],
                                  sem_c).wait()

        issue_idx(0, 0)
        issue_idx(1, 1)
        wait_idx(0, 0)
        issue(0, 0, 0)

        @pl.loop(0, NBLK + 2, step=4)
        def _(t0):
            for u in range(4):
                t = t0 + u
                slot = u & 3
                bf = u & 1

                @pl.when(t < NBLK)
                def _():
                    @pl.when(t + 1 < NBLK)
                    def _():
                        wait_idx(t + 1, (u + 1) & 3)
                        issue(t + 1, (u + 1) & 3, (u + 1) & 1)

                    @pl.when(t + 2 < NBLK)
                    def _():
                        issue_idx(t + 2, (u + 2) & 3)

                    drain(t, slot, bf)
                    m = av2.at[bf]
                    bb = bv2.at[bf]
                    cc = cv2.at[bf]

                    @pl.loop(0, K)
                    def _(i):
                        for j in range(D // 16):
                            sl = pl.ds(j * 16, 16)
                            m[i, sl] = jnp.maximum(
                                m[i, sl] + bb[i, sl] + cc[i, sl], 0.0)

                    pltpu.sync_copy(m, agg.at[ridx.at[slot]], add=True)

        plsc.subcore_barrier()

        pltpu.sync_copy(agg.at[pl.ds(base_row, STRIPE)],
                        out_hbm.at[cid, pl.ds(base_row, STRIPE)])

        @pl.when(sid == NS - 1)
        def _():
            pltpu.sync_copy(agg.at[pl.ds(NS * STRIPE, TAIL)],
                            out_hbm.at[cid, pl.ds(NS * STRIPE, TAIL)])

    return sc_kernel(a, b, c, senders, receivers)


@jax.jit
def kernel(x, edge_index, edge_attr, W_msg, b_msg, W_upd, b_upd):
    assert x.shape == (N, D) and edge_attr.shape == (E, DE)
    w1 = W_msg[:D]
    w2 = W_msg[D:2 * D]
    w3 = W_msg[2 * D:]
    senders = edge_index[0]
    receivers = edge_index[1]

    a, b = _node_prep(x, w1, w2, b_msg)
    c = _edge_prep(edge_attr, w3)
    partials = _sc_edges(a, b, c, senders, receivers)
    return _node_update(x, partials[0], partials[1],
                        W_upd[:D], W_upd[D:], b_upd)


# speedup vs baseline: 5.4803x; 1.0006x over previous
import functools

import jax
import jax.numpy as jnp
from jax import lax
from jax.experimental import pallas as pl
from jax.experimental.pallas import tpu as pltpu
from jax.experimental.pallas import tpu_sc as plsc

N = 10000
E = 320000
D = 128
DE = 16

NC = 2
NS = 16
NW = NC * NS
EPW = E // NW
K = 40
NBLK = EPW // K
STRIPE = 624
TAIL = N - NS * STRIPE


def _prep_body(x_ref, w1_ref, w2_ref, b_ref, a_ref, bout_ref):
    xb = x_ref[...]
    a_ref[...] = jnp.dot(xb, w1_ref[...], preferred_element_type=jnp.float32)
    bout_ref[...] = (
        jnp.dot(xb, w2_ref[...], preferred_element_type=jnp.float32)
        + b_ref[...]
    )


def _node_prep(x, w1, w2, b):
    bn = 2000
    return pl.pallas_call(
        _prep_body,
        grid=(N // bn,),
        in_specs=[
            pl.BlockSpec((bn, D), lambda i: (i, 0)),
            pl.BlockSpec((D, D), lambda i: (0, 0)),
            pl.BlockSpec((D, D), lambda i: (0, 0)),
            pl.BlockSpec((1, D), lambda i: (0, 0)),
        ],
        out_specs=[
            pl.BlockSpec((bn, D), lambda i: (i, 0)),
            pl.BlockSpec((bn, D), lambda i: (i, 0)),
        ],
        out_shape=[
            jax.ShapeDtypeStruct((N, D), jnp.float32),
            jax.ShapeDtypeStruct((N, D), jnp.float32),
        ],
    )(x, w1, w2, b.reshape(1, D))


def _cmsg_body(ea_ref, w3_ref, c_ref):
    c_ref[...] = jnp.dot(ea_ref[...], w3_ref[...],
                         preferred_element_type=jnp.float32)


def _edge_prep(ea, w3):
    be = 4000
    return pl.pallas_call(
        _cmsg_body,
        grid=(E // be,),
        in_specs=[
            pl.BlockSpec((be, DE), lambda i: (i, 0)),
            pl.BlockSpec((DE, D), lambda i: (0, 0)),
        ],
        out_specs=pl.BlockSpec((be, D), lambda i: (i, 0)),
        out_shape=jax.ShapeDtypeStruct((E, D), jnp.float32),
    )(ea, w3)


def _update_body(x_ref, g0_ref, g1_ref, wu1_ref, wu2_ref, b_ref, o_ref):
    xb = x_ref[...]
    agg = g0_ref[...] + g1_ref[...]
    h = (
        jnp.dot(xb, wu1_ref[...], preferred_element_type=jnp.float32)
        + jnp.dot(agg, wu2_ref[...], preferred_element_type=jnp.float32)
        + b_ref[...]
    )
    o_ref[...] = xb + jnp.maximum(h, 0.0)


def _node_update(x, g0, g1, wu1, wu2, b):
    bn = 2000
    return pl.pallas_call(
        _update_body,
        grid=(N // bn,),
        in_specs=[
            pl.BlockSpec((bn, D), lambda i: (i, 0)),
            pl.BlockSpec((bn, D), lambda i: (i, 0)),
            pl.BlockSpec((bn, D), lambda i: (i, 0)),
            pl.BlockSpec((D, D), lambda i: (0, 0)),
            pl.BlockSpec((D, D), lambda i: (0, 0)),
            pl.BlockSpec((1, D), lambda i: (0, 0)),
        ],
        out_specs=pl.BlockSpec((bn, D), lambda i: (i, 0)),
        out_shape=jax.ShapeDtypeStruct((N, D), jnp.float32),
    )(x, g0, g1, wu1, wu2, b.reshape(1, D))


def _sc_edges(a, b, c, senders, receivers):
    mesh = plsc.VectorSubcoreMesh(core_axis_name="c", subcore_axis_name="s")
    @functools.partial(
        pl.kernel,
        mesh=mesh,
        compiler_params=pltpu.CompilerParams(use_tc_tiling_on_sc=True),
        out_type=jax.ShapeDtypeStruct((NC, N, D), jnp.float32),
        scratch_types=[
            pltpu.VMEM((4, K), jnp.int32),
            pltpu.VMEM((4, K), jnp.int32),
            pltpu.VMEM((2, K, D), jnp.float32),
            pltpu.VMEM((2, K, D), jnp.float32),
            pltpu.VMEM((2, K, D), jnp.float32),
            pltpu.VMEM_SHARED((N, D), jnp.float32),
            pltpu.SemaphoreType.DMA,
            pltpu.SemaphoreType.DMA,
            pltpu.SemaphoreType.DMA,
            pltpu.SemaphoreType.DMA,
            pltpu.SemaphoreType.DMA,
        ],
    )
    def sc_kernel(a_hbm, b_hbm, c_hbm, s_hbm, r_hbm, out_hbm,
                  sidx, ridx, av2, bv2, cv2, agg,
                  sem_a, sem_b, sem_c, sem_si, sem_ri):
        cid = lax.axis_index("c")
        sid = lax.axis_index("s")
        wid = sid * NC + cid

        z = av2.at[0]

        @pl.loop(0, K)
        def _(i):
            for j in range(D // 16):
                z[i, pl.ds(j * 16, 16)] = jnp.zeros((16,), jnp.float32)

        base_row = sid * STRIPE
        full, rem = divmod(STRIPE, K)

        @pl.loop(0, full)
        def _(t):
            pltpu.sync_copy(z, agg.at[pl.ds(base_row + t * K, K)])

        if rem:
            pltpu.sync_copy(z.at[pl.ds(0, rem)],
                            agg.at[pl.ds(base_row + full * K, rem)])

        @pl.when(sid == NS - 1)
        def _():
            pltpu.sync_copy(z.at[pl.ds(0, TAIL)],
                            agg.at[pl.ds(NS * STRIPE, TAIL)])

        plsc.subcore_barrier()

        def issue_idx(t, slot):
            ebase = wid * EPW + t * K
            pltpu.async_copy(s_hbm.at[pl.ds(ebase, K)], sidx.at[slot],
                             sem_si)
            pltpu.async_copy(r_hbm.at[pl.ds(ebase, K)], ridx.at[slot],
                             sem_ri)

        def wait_idx(t, slot):
            ebase = wid * EPW + t * K
            pltpu.make_async_copy(s_hbm.at[pl.ds(ebase, K)], sidx.at[slot],
                                  sem_si).wait()
            pltpu.make_async_copy(r_hbm.at[pl.ds(ebase, K)], ridx.at[slot],
                                  sem_ri).wait()

        def issue(t, slot, bf):
            ebase = wid * EPW + t * K
            pltpu.async_copy(a_hbm.at[sidx.at[slot]], av2.at[bf], sem_a)
            pltpu.async_copy(b_hbm.at[ridx.at[slot]], bv2.at[bf], sem_b)
            pltpu.async_copy(c_hbm.at[pl.ds(ebase, K)], cv2.at[bf], sem_c)

        def drain(t, slot, bf):
            ebase = wid * EPW + t * K
            pltpu.make_async_copy(a_hbm.at[sidx.at[slot]], av2.at[bf],
                                  sem_a).wait()
            pltpu.make_async_copy(b_hbm.at[ridx.at[slot]], bv2.at[bf],
                                  sem_b).wait()
            pltpu.make_async_copy(c_hbm.at[pl.ds(ebase, K)], cv2.at[bf],
                                  sem_c).wait()

        issue_idx(0, 0)
        issue_idx(1, 1)
        wait_idx(0, 0)
        issue(0, 0, 0)

        @pl.loop(0, NBLK + 2, step=4)
        def _(t0):
            for u in range(4):
                t = t0 + u
                slot = u & 3
                bf = u & 1

                @pl.when(t < NBLK)
                def _():
                    @pl.when(t + 1 < NBLK)
                    def _():
                        wait_idx(t + 1, (u + 1) & 3)
                        issue(t + 1, (u + 1) & 3, (u + 1) & 1)

                    @pl.when(t + 2 < NBLK)
                    def _():
                        issue_idx(t + 2, (u + 2) & 3)

                    drain(t, slot, bf)
                    m = av2.at[bf]
                    bb = bv2.at[bf]
                    cc = cv2.at[bf]

                    @pl.loop(0, K)
                    def _(i):
                        for j in range(D // 16):
                            sl = pl.ds(j * 16, 16)
                            m[i, sl] = jnp.maximum(
                                m[i, sl] + bb[i, sl] + cc[i, sl], 0.0)

                    pltpu.sync_copy(m, agg.at[ridx.at[slot]], add=True)

        plsc.subcore_barrier()

        pltpu.sync_copy(agg.at[pl.ds(base_row, STRIPE)],
                        out_hbm.at[cid, pl.ds(base_row, STRIPE)])

        @pl.when(sid == NS - 1)
        def _():
            pltpu.sync_copy(agg.at[pl.ds(NS * STRIPE, TAIL)],
                            out_hbm.at[cid, pl.ds(NS * STRIPE, TAIL)])

    return sc_kernel(a, b, c, senders, receivers)


@jax.jit
def kernel(x, edge_index, edge_attr, W_msg, b_msg, W_upd, b_upd):
    assert x.shape == (N, D) and edge_attr.shape == (E, DE)
    w1 = W_msg[:D]
    w2 = W_msg[D:2 * D]
    w3 = W_msg[2 * D:]
    senders = edge_index[0]
    receivers = edge_index[1]

    a, b = _node_prep(x, w1, w2, b_msg)
    c = _edge_prep(edge_attr, w3)
    partials = _sc_edges(a, b, c, senders, receivers)
    return _node_update(x, partials[0], partials[1],
                        W_upd[:D], W_upd[D:], b_upd)


# speedup vs baseline: 5.4827x; 1.0004x over previous
import functools

import jax
import jax.numpy as jnp
from jax import lax
from jax.experimental import pallas as pl
from jax.experimental.pallas import tpu as pltpu
from jax.experimental.pallas import tpu_sc as plsc

N = 10000
E = 320000
D = 128
DE = 16

NC = 2
NS = 16
NW = NC * NS
EPW = E // NW
K = 40
NBLK = EPW // K
STRIPE = 624
TAIL = N - NS * STRIPE


def _prep_body(x_ref, w1_ref, w2_ref, b_ref, a_ref, bout_ref):
    xb = x_ref[...]
    a_ref[...] = jnp.dot(xb, w1_ref[...], preferred_element_type=jnp.float32)
    bout_ref[...] = (
        jnp.dot(xb, w2_ref[...], preferred_element_type=jnp.float32)
        + b_ref[...]
    )


_PAR = pltpu.CompilerParams(dimension_semantics=("parallel",))


def _node_prep(x, w1, w2, b):
    bn = 2000
    return pl.pallas_call(
        _prep_body,
        grid=(N // bn,),
        compiler_params=_PAR,
        in_specs=[
            pl.BlockSpec((bn, D), lambda i: (i, 0)),
            pl.BlockSpec((D, D), lambda i: (0, 0)),
            pl.BlockSpec((D, D), lambda i: (0, 0)),
            pl.BlockSpec((1, D), lambda i: (0, 0)),
        ],
        out_specs=[
            pl.BlockSpec((bn, D), lambda i: (i, 0)),
            pl.BlockSpec((bn, D), lambda i: (i, 0)),
        ],
        out_shape=[
            jax.ShapeDtypeStruct((N, D), jnp.float32),
            jax.ShapeDtypeStruct((N, D), jnp.float32),
        ],
    )(x, w1, w2, b.reshape(1, D))


def _cmsg_body(ea_ref, w3_ref, c_ref):
    c_ref[...] = jnp.dot(ea_ref[...], w3_ref[...],
                         preferred_element_type=jnp.float32)


def _edge_prep(ea, w3):
    be = 4000
    return pl.pallas_call(
        _cmsg_body,
        grid=(E // be,),
        compiler_params=_PAR,
        in_specs=[
            pl.BlockSpec((be, DE), lambda i: (i, 0)),
            pl.BlockSpec((DE, D), lambda i: (0, 0)),
        ],
        out_specs=pl.BlockSpec((be, D), lambda i: (i, 0)),
        out_shape=jax.ShapeDtypeStruct((E, D), jnp.float32),
    )(ea, w3)


def _update_body(x_ref, g0_ref, g1_ref, wu1_ref, wu2_ref, b_ref, o_ref):
    xb = x_ref[...]
    agg = g0_ref[...] + g1_ref[...]
    h = (
        jnp.dot(xb, wu1_ref[...], preferred_element_type=jnp.float32)
        + jnp.dot(agg, wu2_ref[...], preferred_element_type=jnp.float32)
        + b_ref[...]
    )
    o_ref[...] = xb + jnp.maximum(h, 0.0)


def _node_update(x, g0, g1, wu1, wu2, b):
    bn = 2000
    return pl.pallas_call(
        _update_body,
        grid=(N // bn,),
        compiler_params=_PAR,
        in_specs=[
            pl.BlockSpec((bn, D), lambda i: (i, 0)),
            pl.BlockSpec((bn, D), lambda i: (i, 0)),
            pl.BlockSpec((bn, D), lambda i: (i, 0)),
            pl.BlockSpec((D, D), lambda i: (0, 0)),
            pl.BlockSpec((D, D), lambda i: (0, 0)),
            pl.BlockSpec((1, D), lambda i: (0, 0)),
        ],
        out_specs=pl.BlockSpec((bn, D), lambda i: (i, 0)),
        out_shape=jax.ShapeDtypeStruct((N, D), jnp.float32),
    )(x, g0, g1, wu1, wu2, b.reshape(1, D))


def _sc_edges(a, b, c, senders, receivers):
    mesh = plsc.VectorSubcoreMesh(core_axis_name="c", subcore_axis_name="s")
    @functools.partial(
        pl.kernel,
        mesh=mesh,
        compiler_params=pltpu.CompilerParams(use_tc_tiling_on_sc=True),
        out_type=jax.ShapeDtypeStruct((NC, N, D), jnp.float32),
        scratch_types=[
            pltpu.VMEM((4, K), jnp.int32),
            pltpu.VMEM((4, K), jnp.int32),
            pltpu.VMEM((2, K, D), jnp.float32),
            pltpu.VMEM((2, K, D), jnp.float32),
            pltpu.VMEM((2, K, D), jnp.float32),
            pltpu.VMEM_SHARED((N, D), jnp.float32),
            pltpu.SemaphoreType.DMA,
            pltpu.SemaphoreType.DMA,
            pltpu.SemaphoreType.DMA,
            pltpu.SemaphoreType.DMA,
            pltpu.SemaphoreType.DMA,
        ],
    )
    def sc_kernel(a_hbm, b_hbm, c_hbm, s_hbm, r_hbm, out_hbm,
                  sidx, ridx, av2, bv2, cv2, agg,
                  sem_a, sem_b, sem_c, sem_si, sem_ri):
        cid = lax.axis_index("c")
        sid = lax.axis_index("s")
        wid = sid * NC + cid

        z = av2.at[0]

        @pl.loop(0, K)
        def _(i):
            for j in range(D // 16):
                z[i, pl.ds(j * 16, 16)] = jnp.zeros((16,), jnp.float32)

        base_row = sid * STRIPE
        full, rem = divmod(STRIPE, K)

        @pl.loop(0, full)
        def _(t):
            pltpu.sync_copy(z, agg.at[pl.ds(base_row + t * K, K)])

        if rem:
            pltpu.sync_copy(z.at[pl.ds(0, rem)],
                            agg.at[pl.ds(base_row + full * K, rem)])

        @pl.when(sid == NS - 1)
        def _():
            pltpu.sync_copy(z.at[pl.ds(0, TAIL)],
                            agg.at[pl.ds(NS * STRIPE, TAIL)])

        plsc.subcore_barrier()

        def issue_idx(t, slot):
            ebase = wid * EPW + t * K
            pltpu.async_copy(s_hbm.at[pl.ds(ebase, K)], sidx.at[slot],
                             sem_si)
            pltpu.async_copy(r_hbm.at[pl.ds(ebase, K)], ridx.at[slot],
                             sem_ri)

        def wait_idx(t, slot):
            ebase = wid * EPW + t * K
            pltpu.make_async_copy(s_hbm.at[pl.ds(ebase, K)], sidx.at[slot],
                                  sem_si).wait()
            pltpu.make_async_copy(r_hbm.at[pl.ds(ebase, K)], ridx.at[slot],
                                  sem_ri).wait()

        def issue(t, slot, bf):
            ebase = wid * EPW + t * K
            pltpu.async_copy(a_hbm.at[sidx.at[slot]], av2.at[bf], sem_a)
            pltpu.async_copy(b_hbm.at[ridx.at[slot]], bv2.at[bf], sem_b)
            pltpu.async_copy(c_hbm.at[pl.ds(ebase, K)], cv2.at[bf], sem_c)

        def drain(t, slot, bf):
            ebase = wid * EPW + t * K
            pltpu.make_async_copy(a_hbm.at[sidx.at[slot]], av2.at[bf],
                                  sem_a).wait()
            pltpu.make_async_copy(b_hbm.at[ridx.at[slot]], bv2.at[bf],
                                  sem_b).wait()
            pltpu.make_async_copy(c_hbm.at[pl.ds(ebase, K)], cv2.at[bf],
                                  sem_c).wait()

        issue_idx(0, 0)
        issue_idx(1, 1)
        wait_idx(0, 0)
        issue(0, 0, 0)

        @pl.loop(0, NBLK + 2, step=4)
        def _(t0):
            for u in range(4):
                t = t0 + u
                slot = u & 3
                bf = u & 1

                @pl.when(t < NBLK)
                def _():
                    @pl.when(t + 1 < NBLK)
                    def _():
                        wait_idx(t + 1, (u + 1) & 3)
                        issue(t + 1, (u + 1) & 3, (u + 1) & 1)

                    @pl.when(t + 2 < NBLK)
                    def _():
                        issue_idx(t + 2, (u + 2) & 3)

                    drain(t, slot, bf)
                    m = av2.at[bf]
                    bb = bv2.at[bf]
                    cc = cv2.at[bf]

                    @pl.loop(0, K)
                    def _(i):
                        for j in range(D // 16):
                            sl = pl.ds(j * 16, 16)
                            m[i, sl] = jnp.maximum(
                                m[i, sl] + bb[i, sl] + cc[i, sl], 0.0)

                    pltpu.sync_copy(m, agg.at[ridx.at[slot]], add=True)

        plsc.subcore_barrier()

        pltpu.sync_copy(agg.at[pl.ds(base_row, STRIPE)],
                        out_hbm.at[cid, pl.ds(base_row, STRIPE)])

        @pl.when(sid == NS - 1)
        def _():
            pltpu.sync_copy(agg.at[pl.ds(NS * STRIPE, TAIL)],
                            out_hbm.at[cid, pl.ds(NS * STRIPE, TAIL)])

    return sc_kernel(a, b, c, senders, receivers)


@jax.jit
def kernel(x, edge_index, edge_attr, W_msg, b_msg, W_upd, b_upd):
    assert x.shape == (N, D) and edge_attr.shape == (E, DE)
    w1 = W_msg[:D]
    w2 = W_msg[D:2 * D]
    w3 = W_msg[2 * D:]
    senders = edge_index[0]
    receivers = edge_index[1]

    a, b = _node_prep(x, w1, w2, b_msg)
    c = _edge_prep(edge_attr, w3)
    partials = _sc_edges(a, b, c, senders, receivers)
    return _node_update(x, partials[0], partials[1],
                        W_upd[:D], W_upd[D:], b_upd)


# speedup vs baseline: 7.2871x; 1.3291x over previous
import functools

import jax
import jax.numpy as jnp
from jax import lax
from jax.experimental import pallas as pl
from jax.experimental.pallas import tpu as pltpu
from jax.experimental.pallas import tpu_sc as plsc

N = 10000
E = 320000
D = 128
DE = 16

NC = 2
NS = 16
NW = NC * NS
EPW = E // NW
K = 40
NBLK = EPW // K
STRIPE = 624
TAIL = N - NS * STRIPE


def _prep_body(x_ref, w1_ref, w2_ref, b_ref, a_ref, bout_ref):
    xb = x_ref[...]
    a_ref[...] = jnp.dot(xb, w1_ref[...], preferred_element_type=jnp.float32)
    bout_ref[...] = (
        jnp.dot(xb, w2_ref[...], preferred_element_type=jnp.float32)
        + b_ref[...]
    )


_PAR = pltpu.CompilerParams(dimension_semantics=("parallel",))


def _node_prep(x, w1, w2, b):
    bn = 2000
    return pl.pallas_call(
        _prep_body,
        grid=(N // bn,),
        compiler_params=_PAR,
        in_specs=[
            pl.BlockSpec((bn, D), lambda i: (i, 0)),
            pl.BlockSpec((D, D), lambda i: (0, 0)),
            pl.BlockSpec((D, D), lambda i: (0, 0)),
            pl.BlockSpec((1, D), lambda i: (0, 0)),
        ],
        out_specs=[
            pl.BlockSpec((bn, D), lambda i: (i, 0)),
            pl.BlockSpec((bn, D), lambda i: (i, 0)),
        ],
        out_shape=[
            jax.ShapeDtypeStruct((N, D), jnp.float32),
            jax.ShapeDtypeStruct((N, D), jnp.float32),
        ],
    )(x, w1, w2, b.reshape(1, D))


def _cmsg_body(eat_ref, w3_ref, c_ref):
    c_ref[...] = jax.lax.dot_general(
        eat_ref[...], w3_ref[...],
        dimension_numbers=(((0,), (0,)), ((), ())),
        preferred_element_type=jnp.float32)


def _edge_prep(ea, w3):
    be = 12800
    ea_t = ea.T
    return pl.pallas_call(
        _cmsg_body,
        grid=(E // be,),
        compiler_params=_PAR,
        in_specs=[
            pl.BlockSpec((DE, be), lambda i: (0, i)),
            pl.BlockSpec((DE, D), lambda i: (0, 0)),
        ],
        out_specs=pl.BlockSpec((be, D), lambda i: (i, 0)),
        out_shape=jax.ShapeDtypeStruct((E, D), jnp.float32),
    )(ea_t, w3)


def _update_body(x_ref, g0_ref, g1_ref, wu1_ref, wu2_ref, b_ref, o_ref):
    xb = x_ref[...]
    agg = g0_ref[...] + g1_ref[...]
    h = (
        jnp.dot(xb, wu1_ref[...], preferred_element_type=jnp.float32)
        + jnp.dot(agg, wu2_ref[...], preferred_element_type=jnp.float32)
        + b_ref[...]
    )
    o_ref[...] = xb + jnp.maximum(h, 0.0)


def _node_update(x, g0, g1, wu1, wu2, b):
    bn = 2000
    return pl.pallas_call(
        _update_body,
        grid=(N // bn,),
        compiler_params=_PAR,
        in_specs=[
            pl.BlockSpec((bn, D), lambda i: (i, 0)),
            pl.BlockSpec((bn, D), lambda i: (i, 0)),
            pl.BlockSpec((bn, D), lambda i: (i, 0)),
            pl.BlockSpec((D, D), lambda i: (0, 0)),
            pl.BlockSpec((D, D), lambda i: (0, 0)),
            pl.BlockSpec((1, D), lambda i: (0, 0)),
        ],
        out_specs=pl.BlockSpec((bn, D), lambda i: (i, 0)),
        out_shape=jax.ShapeDtypeStruct((N, D), jnp.float32),
    )(x, g0, g1, wu1, wu2, b.reshape(1, D))


def _sc_edges(a, b, c, senders, receivers):
    mesh = plsc.VectorSubcoreMesh(core_axis_name="c", subcore_axis_name="s")
    @functools.partial(
        pl.kernel,
        mesh=mesh,
        compiler_params=pltpu.CompilerParams(use_tc_tiling_on_sc=True),
        out_type=jax.ShapeDtypeStruct((NC, N, D), jnp.float32),
        scratch_types=[
            pltpu.VMEM((4, K), jnp.int32),
            pltpu.VMEM((4, K), jnp.int32),
            pltpu.VMEM((2, K, D), jnp.float32),
            pltpu.VMEM((2, K, D), jnp.float32),
            pltpu.VMEM((2, K, D), jnp.float32),
            pltpu.VMEM_SHARED((N, D), jnp.float32),
            pltpu.SemaphoreType.DMA,
            pltpu.SemaphoreType.DMA,
            pltpu.SemaphoreType.DMA,
            pltpu.SemaphoreType.DMA,
            pltpu.SemaphoreType.DMA,
        ],
    )
    def sc_kernel(a_hbm, b_hbm, c_hbm, s_hbm, r_hbm, out_hbm,
                  sidx, ridx, av2, bv2, cv2, agg,
                  sem_a, sem_b, sem_c, sem_si, sem_ri):
        cid = lax.axis_index("c")
        sid = lax.axis_index("s")
        wid = sid * NC + cid

        z = av2.at[0]

        @pl.loop(0, K)
        def _(i):
            for j in range(D // 16):
                z[i, pl.ds(j * 16, 16)] = jnp.zeros((16,), jnp.float32)

        base_row = sid * STRIPE
        full, rem = divmod(STRIPE, K)

        @pl.loop(0, full)
        def _(t):
            pltpu.sync_copy(z, agg.at[pl.ds(base_row + t * K, K)])

        if rem:
            pltpu.sync_copy(z.at[pl.ds(0, rem)],
                            agg.at[pl.ds(base_row + full * K, rem)])

        @pl.when(sid == NS - 1)
        def _():
            pltpu.sync_copy(z.at[pl.ds(0, TAIL)],
                            agg.at[pl.ds(NS * STRIPE, TAIL)])

        plsc.subcore_barrier()

        def issue_idx(t, slot):
            ebase = wid * EPW + t * K
            pltpu.async_copy(s_hbm.at[pl.ds(ebase, K)], sidx.at[slot],
                             sem_si)
            pltpu.async_copy(r_hbm.at[pl.ds(ebase, K)], ridx.at[slot],
                             sem_ri)

        def wait_idx(t, slot):
            ebase = wid * EPW + t * K
            pltpu.make_async_copy(s_hbm.at[pl.ds(ebase, K)], sidx.at[slot],
                                  sem_si).wait()
            pltpu.make_async_copy(r_hbm.at[pl.ds(ebase, K)], ridx.at[slot],
                                  sem_ri).wait()

        def issue(t, slot, bf):
            ebase = wid * EPW + t * K
            pltpu.async_copy(a_hbm.at[sidx.at[slot]], av2.at[bf], sem_a)
            pltpu.async_copy(b_hbm.at[ridx.at[slot]], bv2.at[bf], sem_b)
            pltpu.async_copy(c_hbm.at[pl.ds(ebase, K)], cv2.at[bf], sem_c)

        def drain(t, slot, bf):
            ebase = wid * EPW + t * K
            pltpu.make_async_copy(a_hbm.at[sidx.at[slot]], av2.at[bf],
                                  sem_a).wait()
            pltpu.make_async_copy(b_hbm.at[ridx.at[slot]], bv2.at[bf],
                                  sem_b).wait()
            pltpu.make_async_copy(c_hbm.at[pl.ds(ebase, K)], cv2.at[bf],
                                  sem_c).wait()

        issue_idx(0, 0)
        issue_idx(1, 1)
        wait_idx(0, 0)
        issue(0, 0, 0)

        @pl.loop(0, NBLK + 2, step=4)
        def _(t0):
            for u in range(4):
                t = t0 + u
                slot = u & 3
                bf = u & 1

                @pl.when(t < NBLK)
                def _():
                    @pl.when(t + 1 < NBLK)
                    def _():
                        wait_idx(t + 1, (u + 1) & 3)
                        issue(t + 1, (u + 1) & 3, (u + 1) & 1)

                    @pl.when(t + 2 < NBLK)
                    def _():
                        issue_idx(t + 2, (u + 2) & 3)

                    drain(t, slot, bf)
                    m = av2.at[bf]
                    bb = bv2.at[bf]
                    cc = cv2.at[bf]

                    @pl.loop(0, K)
                    def _(i):
                        for j in range(D // 16):
                            sl = pl.ds(j * 16, 16)
                            m[i, sl] = jnp.maximum(
                                m[i, sl] + bb[i, sl] + cc[i, sl], 0.0)

                    pltpu.sync_copy(m, agg.at[ridx.at[slot]], add=True)

        plsc.subcore_barrier()

        pltpu.sync_copy(agg.at[pl.ds(base_row, STRIPE)],
                        out_hbm.at[cid, pl.ds(base_row, STRIPE)])

        @pl.when(sid == NS - 1)
        def _():
            pltpu.sync_copy(agg.at[pl.ds(NS * STRIPE, TAIL)],
                            out_hbm.at[cid, pl.ds(NS * STRIPE, TAIL)])

    return sc_kernel(a, b, c, senders, receivers)


@jax.jit
def kernel(x, edge_index, edge_attr, W_msg, b_msg, W_upd, b_upd):
    assert x.shape == (N, D) and edge_attr.shape == (E, DE)
    w1 = W_msg[:D]
    w2 = W_msg[D:2 * D]
    w3 = W_msg[2 * D:]
    senders = edge_index[0]
    receivers = edge_index[1]

    a, b = _node_prep(x, w1, w2, b_msg)
    c = _edge_prep(edge_attr, w3)
    partials = _sc_edges(a, b, c, senders, receivers)
    return _node_update(x, partials[0], partials[1],
                        W_upd[:D], W_upd[D:], b_upd)
